# baseline (speedup 1.0000x reference)
import jax
import jax.numpy as jnp
from jax.experimental import pallas as pl
from jax.experimental.pallas import tpu as pltpu

_BPS = 8
_PS = 1


def _vq_body(ze_ref, emb_ref, zq_ref, idx_ref, loss_ref, enorm_ref):
    g = pl.program_id(0)
    j = pl.program_id(1)
    ng = pl.num_programs(0)
    nj = pl.num_programs(1)
    emb = emb_ref[...]
    C = emb.shape[0]
    D = ze_ref.shape[1]
    PC = ze_ref.shape[2]

    @pl.when((g == 0) & (j == 0))
    def _():
        enorm_ref[...] = jnp.sum(emb * emb, axis=1, keepdims=True)

    part = jnp.zeros((1, 1), jnp.float32)
    for i in range(_BPS):
        ze = ze_ref[i]
        scores = jnp.dot(emb, ze, preferred_element_type=jnp.float32)
        fnorm = jnp.sum(ze * ze, axis=0, keepdims=True)
        dist = (fnorm + enorm_ref[...]) - 2.0 * scores
        idx = jnp.argmin(dist, axis=0)
        onehot = (jax.lax.broadcasted_iota(jnp.int32, (C, PC), 0)
                  == idx[None, :]).astype(jnp.float32)
        zq = jax.lax.dot_general(emb, onehot, (((0,), (0,)), ((), ())),
                                 preferred_element_type=jnp.float32)
        zq_ref[i] = zq
        idx_ref[i] = idx.reshape(1, PC).astype(jnp.int32)
        diff = ze - zq
        part = part + jnp.sum(diff * diff).reshape(1, 1)

    @pl.when((g == 0) & (j == 0))
    def _():
        loss_ref[...] = part

    @pl.when((g != 0) | (j != 0))
    def _():
        loss_ref[...] += part

    @pl.when((g == ng - 1) & (j == nj - 1))
    def _():
        loss_ref[...] = loss_ref[...] / (ng * _BPS * D * nj * PC)


def kernel(z_e, embed):
    B, D, H, W = z_e.shape
    P = H * W
    PC = P // _PS
    C = embed.shape[0]
    ze = z_e.reshape(B, D, P)

    zq, idx, loss = pl.pallas_call(
        _vq_body,
        grid=(B // _BPS, _PS),
        in_specs=[
            pl.BlockSpec((_BPS, D, PC), lambda g, j: (g, 0, j)),
            pl.BlockSpec((C, D), lambda g, j: (0, 0)),
        ],
        out_specs=[
            pl.BlockSpec((_BPS, D, PC), lambda g, j: (g, 0, j)),
            pl.BlockSpec((_BPS, 1, PC), lambda g, j: (g, 0, j)),
            pl.BlockSpec((1, 1), lambda g, j: (0, 0)),
        ],
        out_shape=[
            jax.ShapeDtypeStruct((B, D, P), jnp.float32),
            jax.ShapeDtypeStruct((B, 1, P), jnp.int32),
            jax.ShapeDtypeStruct((1, 1), jnp.float32),
        ],
        scratch_shapes=[pltpu.VMEM((C, 1), jnp.float32)],
    )(ze, embed)

    z_q_st = zq.reshape(B, D, H, W)
    commitment_loss = loss.reshape(())
    indices_out = idx.reshape(B, H, W)
    return (z_q_st, commitment_loss, indices_out)
